# baseline (speedup 1.0000x reference)
import functools

import jax
import jax.numpy as jnp
from jax import lax
from jax.experimental import pallas as pl
from jax.experimental.pallas import tpu as pltpu
from jax.experimental.pallas import tpu_sc as plsc

NC = 2
NS = 16
NW = NC * NS
CH = 512
CR = CH // 128


def _make_deg(EP, NP):
  ept = EP // NW
  nsup = ept // (2 * CH)
  rpt = NP // NS
  mesh = plsc.VectorSubcoreMesh(core_axis_name="c", subcore_axis_name="s",
                                num_cores=NC, num_subcores=NS)

  @functools.partial(
      pl.kernel,
      out_type=(jax.ShapeDtypeStruct((NP,), jnp.float32),
                jax.ShapeDtypeStruct((NP,), jnp.float32)),
      mesh=mesh,
      scratch_types=[
          pltpu.VMEM((CH,), jnp.int32),
          pltpu.VMEM((CH,), jnp.int32),
          pltpu.VMEM((CH,), jnp.float32),
          pltpu.VMEM((rpt,), jnp.float32),
          pltpu.VMEM_SHARED((NP,), jnp.float32),
          pltpu.SemaphoreType.DMA,
          pltpu.SemaphoreType.DMA,
          pltpu.SemaphoreType.DMA,
          pltpu.SemaphoreType.DMA,
      ],
      compiler_params=pltpu.CompilerParams(use_tc_tiling_on_sc=False),
  )
  def deg_k(dst_hbm, out0, out1, dst_v0, dst_v1, ones_v, bounce, deg_sh,
            ssem0, ssem1, lsem0, lsem1):
    c = lax.axis_index("c")
    s = lax.axis_index("s")
    wid = c * NS + s

    def fill_ones(i, _):
      ones_v[pl.ds(i * 16, 16)] = jnp.ones((16,), jnp.float32)
      return _
    lax.fori_loop(0, CH // 16, fill_ones, None)

    def fill_zero(i, _):
      bounce[pl.ds(i * 16, 16)] = jnp.zeros((16,), jnp.float32)
      return _
    lax.fori_loop(0, rpt // 16, fill_zero, None)

    pltpu.sync_copy(bounce, deg_sh.at[pl.ds(s * rpt, rpt)])
    plsc.subcore_barrier()

    def sup(q, _):
      e0 = wid * ept + q * 2 * CH

      @pl.when(q > 0)
      def _():
        pltpu.make_async_copy(ones_v, deg_sh.at[dst_v0], ssem0).wait()
      l0 = pltpu.async_copy(dst_hbm.at[pl.ds(e0, CH)], dst_v0, lsem0)

      @pl.when(q > 0)
      def _():
        pltpu.make_async_copy(ones_v, deg_sh.at[dst_v1], ssem1).wait()
      l1 = pltpu.async_copy(dst_hbm.at[pl.ds(e0 + CH, CH)], dst_v1, lsem1)

      l0.wait()
      pltpu.async_copy(ones_v, deg_sh.at[dst_v0], ssem0, add=True)
      l1.wait()
      pltpu.async_copy(ones_v, deg_sh.at[dst_v1], ssem1, add=True)
      return _
    lax.fori_loop(0, nsup, sup, None)

    pltpu.make_async_copy(ones_v, deg_sh.at[dst_v0], ssem0).wait()
    pltpu.make_async_copy(ones_v, deg_sh.at[dst_v1], ssem1).wait()
    plsc.subcore_barrier()

    pltpu.sync_copy(deg_sh.at[pl.ds(s * rpt, rpt)], bounce)

    @pl.when(c == 0)
    def _():
      pltpu.sync_copy(bounce, out0.at[pl.ds(s * rpt, rpt)])

    @pl.when(c == 1)
    def _():
      pltpu.sync_copy(bounce, out1.at[pl.ds(s * rpt, rpt)])

  return deg_k


def _make_agg(EP, NP, F_OUT):
  ept = EP // NW
  nsup = ept // (2 * CH)
  rpt = NP // NS
  mesh = plsc.VectorSubcoreMesh(core_axis_name="c", subcore_axis_name="s",
                                num_cores=NC, num_subcores=NS)

  @functools.partial(
      pl.kernel,
      out_type=(jax.ShapeDtypeStruct((NP, F_OUT), jnp.float32),
                jax.ShapeDtypeStruct((NP, F_OUT), jnp.float32)),
      mesh=mesh,
      scratch_types=[
          pltpu.VMEM((CH,), jnp.int32),
          pltpu.VMEM((CH,), jnp.int32),
          pltpu.VMEM((CH,), jnp.int32),
          pltpu.VMEM((CH,), jnp.int32),
          pltpu.VMEM((CH, F_OUT), jnp.float32),
          pltpu.VMEM((CH, F_OUT), jnp.float32),
          pltpu.VMEM_SHARED((NP, F_OUT), jnp.float32),
          pltpu.VMEM_SHARED((NP, F_OUT), jnp.float32),
          pltpu.SemaphoreType.DMA,
          pltpu.SemaphoreType.DMA,
          pltpu.SemaphoreType.DMA,
          pltpu.SemaphoreType.DMA,
          pltpu.SemaphoreType.DMA,
          pltpu.SemaphoreType.DMA,
      ],
      compiler_params=pltpu.CompilerParams(use_tc_tiling_on_sc=False),
  )
  def agg_k(src_hbm, dst_hbm, h2_hbm, out0, out1,
            src_v0, dst_v0, src_v1, dst_v1, rows_v0, rows_v1, s_sh, h2_sh,
            gsem0, gsem1, ssem0, ssem1, lsem0, lsem1):
    bounce = rows_v0
    c = lax.axis_index("c")
    s = lax.axis_index("s")
    wid = c * NS + s

    def fill_zero(i, _):
      bounce[i] = jnp.zeros((F_OUT,), jnp.float32)
      return _
    lax.fori_loop(0, CH, fill_zero, None)

    pieces = [(z * CH, CH) for z in range(rpt // CH)]
    if rpt % CH:
      pieces.append((rpt - rpt % CH, rpt % CH))
    for off, ln in pieces:
      pltpu.sync_copy(bounce.at[pl.ds(0, ln)],
                      s_sh.at[pl.ds(s * rpt + off, ln)])
    for off, ln in pieces:
      pltpu.sync_copy(h2_hbm.at[pl.ds(s * rpt + off, ln)],
                      rows_v1.at[pl.ds(0, ln)])
      pltpu.sync_copy(rows_v1.at[pl.ds(0, ln)],
                      h2_sh.at[pl.ds(s * rpt + off, ln)])
    plsc.subcore_barrier()

    def sup(q, _):
      e0 = wid * ept + q * 2 * CH

      ls0 = pltpu.async_copy(src_hbm.at[pl.ds(e0, CH)], src_v0, lsem0)
      ls1 = pltpu.async_copy(src_hbm.at[pl.ds(e0 + CH, CH)], src_v1, lsem1)

      @pl.when(q > 0)
      def _():
        pltpu.make_async_copy(rows_v0, s_sh.at[dst_v0], ssem0).wait()
      ld0 = pltpu.async_copy(dst_hbm.at[pl.ds(e0, CH)], dst_v0, lsem0)

      @pl.when(q > 0)
      def _():
        pltpu.make_async_copy(rows_v1, s_sh.at[dst_v1], ssem1).wait()
      ld1 = pltpu.async_copy(dst_hbm.at[pl.ds(e0 + CH, CH)], dst_v1, lsem1)

      ls0.wait()
      ld0.wait()
      g0 = pltpu.async_copy(h2_sh.at[src_v0], rows_v0, gsem0)
      ls1.wait()
      ld1.wait()
      g1 = pltpu.async_copy(h2_hbm.at[src_v1], rows_v1, gsem1)

      g0.wait()
      pltpu.async_copy(rows_v0, s_sh.at[dst_v0], ssem0, add=True)
      g1.wait()
      pltpu.async_copy(rows_v1, s_sh.at[dst_v1], ssem1, add=True)
      return _
    lax.fori_loop(0, nsup, sup, None)

    pltpu.make_async_copy(rows_v0, s_sh.at[dst_v0], ssem0).wait()
    pltpu.make_async_copy(rows_v1, s_sh.at[dst_v1], ssem1).wait()
    plsc.subcore_barrier()

    for off, ln in pieces:
      pltpu.sync_copy(s_sh.at[pl.ds(s * rpt + off, ln)],
                      bounce.at[pl.ds(0, ln)])

      @pl.when(c == 0)
      def _():
        pltpu.sync_copy(bounce.at[pl.ds(0, ln)],
                        out0.at[pl.ds(s * rpt + off, ln)])

      @pl.when(c == 1)
      def _():
        pltpu.sync_copy(bounce.at[pl.ds(0, ln)],
                        out1.at[pl.ds(s * rpt + off, ln)])

  return agg_k


def _mm_body(xt_ref, w_ref, h_ref):
  h_ref[...] = lax.dot_general(
      xt_ref[...], w_ref[...],
      dimension_numbers=(((0,), (0,)), ((), ())),
      preferred_element_type=jnp.float32)


def _scale_body(h_ref, d0_ref, d1_ref, h2_ref):
  dis = lax.rsqrt(d0_ref[...] + d1_ref[...] + 1.0)
  h2_ref[...] = h_ref[...] * dis[:, None]


def _fin_body(s0_ref, s1_ref, h2_ref, d0_ref, d1_ref, b_ref, o_ref):
  dis = lax.rsqrt(d0_ref[...] + d1_ref[...] + 1.0)
  t = s0_ref[...] + s1_ref[...] + h2_ref[...]
  o_ref[...] = jnp.maximum(dis[:, None] * t + b_ref[...], 0.0)


def kernel(X, edge_index, W1, b1):
  N, F_IN = X.shape
  F_OUT = W1.shape[1]
  E = edge_index.shape[1]

  align = 2 * CH * NW
  EP = -(-E // align) * align
  NP = -(-N // 256) * 256

  src = edge_index[0]
  dst = edge_index[1]
  pad = EP - E
  if pad:
    pad_i = jnp.arange(pad, dtype=jnp.int32)
    src = jnp.concatenate([src, pad_i % N])
    dst = jnp.concatenate([dst, N + pad_i % (NP - N)])
  deg0, deg1 = _make_deg(EP, NP)(dst)

  BN = 2048
  nb = -(-N // BN)
  h = pl.pallas_call(
      _mm_body,
      grid=(nb,),
      in_specs=[
          pl.BlockSpec((F_IN, BN), lambda i: (0, i)),
          pl.BlockSpec((F_IN, F_OUT), lambda i: (0, 0)),
      ],
      out_specs=pl.BlockSpec((BN, F_OUT), lambda i: (i, 0)),
      out_shape=jax.ShapeDtypeStruct((N, F_OUT), jnp.float32),
  )(X.T, W1)

  BE = 8192
  ne = -(-NP // BE)
  h2 = pl.pallas_call(
      _scale_body,
      grid=(ne,),
      in_specs=[
          pl.BlockSpec((BE, F_OUT), lambda i: (i, 0)),
          pl.BlockSpec((BE,), lambda i: (i,)),
          pl.BlockSpec((BE,), lambda i: (i,)),
      ],
      out_specs=pl.BlockSpec((BE, F_OUT), lambda i: (i, 0)),
      out_shape=jax.ShapeDtypeStruct((NP, F_OUT), jnp.float32),
  )(h, deg0, deg1)

  s0, s1 = _make_agg(EP, NP, F_OUT)(src, dst, h2)

  out = pl.pallas_call(
      _fin_body,
      grid=(ne,),
      in_specs=[
          pl.BlockSpec((BE, F_OUT), lambda i: (i, 0)),
          pl.BlockSpec((BE, F_OUT), lambda i: (i, 0)),
          pl.BlockSpec((BE, F_OUT), lambda i: (i, 0)),
          pl.BlockSpec((BE,), lambda i: (i,)),
          pl.BlockSpec((BE,), lambda i: (i,)),
          pl.BlockSpec((1, F_OUT), lambda i: (0, 0)),
      ],
      out_specs=pl.BlockSpec((BE, F_OUT), lambda i: (i, 0)),
      out_shape=jax.ShapeDtypeStruct((N, F_OUT), jnp.float32),
  )(s0, s1, h2, deg0, deg1, b1[None, :])

  return out


# speedup vs baseline: 1.2284x; 1.2284x over previous
import functools

import jax
import jax.numpy as jnp
from jax import lax
from jax.experimental import pallas as pl
from jax.experimental.pallas import tpu as pltpu
from jax.experimental.pallas import tpu_sc as plsc

NC = 2
NS = 16
NW = NC * NS
CH = 2048
CRr = CH // 128


def _chunking(R):
  full_chunks = R // CRr
  rem_rows = R - full_chunks * CRr
  base = full_chunks // NW
  extra = full_chunks - base * NW
  assert base % 2 == 0, "need an even number of guaranteed chunks per tile"
  return full_chunks, rem_rows, base, extra


def _make_deg(R, NP):
  full_chunks, rem_rows, base, extra = _chunking(R)
  nsup = base // 2
  rpt = NP // NS
  mesh = plsc.VectorSubcoreMesh(core_axis_name="c", subcore_axis_name="s",
                                num_cores=NC, num_subcores=NS)

  @functools.partial(
      pl.kernel,
      out_type=(jax.ShapeDtypeStruct((NP,), jnp.float32),
                jax.ShapeDtypeStruct((NP,), jnp.float32)),
      mesh=mesh,
      scratch_types=[
          pltpu.VMEM((CRr, 2, 128), jnp.int32),
          pltpu.VMEM((CRr, 2, 128), jnp.int32),
          pltpu.VMEM((128,), jnp.float32),
          pltpu.VMEM((rpt,), jnp.float32),
          pltpu.VMEM_SHARED((NP,), jnp.float32),
          pltpu.SemaphoreType.DMA,
          pltpu.SemaphoreType.DMA,
          pltpu.SemaphoreType.DMA,
          pltpu.SemaphoreType.DMA,
      ],
      compiler_params=pltpu.CompilerParams(use_tc_tiling_on_sc=False),
  )
  def deg_k(ei_hbm, out0, out1, pb0, pb1, ones_v, bounce, deg_sh,
            ssem0, ssem1, lsem0, lsem1):
    c = lax.axis_index("c")
    s = lax.axis_index("s")
    wid = c * NS + s

    def fill_ones(i, _):
      ones_v[pl.ds(i * 16, 16)] = jnp.ones((16,), jnp.float32)
      return _
    lax.fori_loop(0, 128 // 16, fill_ones, None)

    def fill_zero(i, _):
      bounce[pl.ds(i * 16, 16)] = jnp.zeros((16,), jnp.float32)
      return _
    lax.fori_loop(0, rpt // 16, fill_zero, None)

    pltpu.sync_copy(bounce, deg_sh.at[pl.ds(s * rpt, rpt)])
    plsc.subcore_barrier()

    def scat(pb, sem, n=CRr):
      for k in range(n):
        pltpu.async_copy(ones_v, deg_sh.at[pb.at[k, 1]], sem, add=True)

    def drain(pb, sem, n=CRr):
      for k in range(n):
        pltpu.make_async_copy(ones_v, deg_sh.at[pb.at[k, 1]], sem).wait()

    def sup(q, _):
      g0 = (2 * q) * NW + wid
      g1 = (2 * q + 1) * NW + wid

      @pl.when(q > 0)
      def _():
        drain(pb0, ssem0)
      l0 = pltpu.async_copy(ei_hbm.at[pl.ds(g0 * CRr, CRr)], pb0, lsem0)

      @pl.when(q > 0)
      def _():
        drain(pb1, ssem1)
      l1 = pltpu.async_copy(ei_hbm.at[pl.ds(g1 * CRr, CRr)], pb1, lsem1)

      l0.wait()
      scat(pb0, ssem0)
      l1.wait()
      scat(pb1, ssem1)
      return _
    lax.fori_loop(0, nsup, sup, None)

    @pl.when(wid < extra)
    def _():
      drain(pb0, ssem0)
      pltpu.sync_copy(ei_hbm.at[pl.ds((base * NW + wid) * CRr, CRr)], pb0)
      scat(pb0, ssem0)

    if rem_rows:
      @pl.when(wid == NW - 1)
      def _():
        drain(pb1, ssem1)
        pltpu.sync_copy(ei_hbm.at[pl.ds(full_chunks * CRr, rem_rows)],
                        pb1.at[pl.ds(0, rem_rows)])
        scat(pb1, ssem1, rem_rows)

    drain(pb0, ssem0)
    if rem_rows:
      @pl.when(wid == NW - 1)
      def _():
        drain(pb1, ssem1, rem_rows)

      @pl.when(wid != NW - 1)
      def _():
        drain(pb1, ssem1)
    else:
      drain(pb1, ssem1)
    plsc.subcore_barrier()

    pltpu.sync_copy(deg_sh.at[pl.ds(s * rpt, rpt)], bounce)

    @pl.when(c == 0)
    def _():
      pltpu.sync_copy(bounce, out0.at[pl.ds(s * rpt, rpt)])

    @pl.when(c == 1)
    def _():
      pltpu.sync_copy(bounce, out1.at[pl.ds(s * rpt, rpt)])

  return deg_k


def _make_agg(R, NP, F_OUT):
  full_chunks, rem_rows, base, extra = _chunking(R)
  nsup = base // 2
  rpt = NP // NS
  mesh = plsc.VectorSubcoreMesh(core_axis_name="c", subcore_axis_name="s",
                                num_cores=NC, num_subcores=NS)

  @functools.partial(
      pl.kernel,
      out_type=(jax.ShapeDtypeStruct((NP, F_OUT), jnp.float32),
                jax.ShapeDtypeStruct((NP, F_OUT), jnp.float32)),
      mesh=mesh,
      scratch_types=[
          pltpu.VMEM((CRr, 2, 128), jnp.int32),
          pltpu.VMEM((CRr, 2, 128), jnp.int32),
          pltpu.VMEM((CH, F_OUT), jnp.float32),
          pltpu.VMEM((CH, F_OUT), jnp.float32),
          pltpu.VMEM_SHARED((NP, F_OUT), jnp.float32),
          pltpu.SemaphoreType.DMA,
          pltpu.SemaphoreType.DMA,
          pltpu.SemaphoreType.DMA,
          pltpu.SemaphoreType.DMA,
          pltpu.SemaphoreType.DMA,
          pltpu.SemaphoreType.DMA,
      ],
      compiler_params=pltpu.CompilerParams(use_tc_tiling_on_sc=False),
  )
  def agg_k(ei_hbm, h2_hbm, out0, out1,
            pb0, pb1, rows_v0, rows_v1, s_sh,
            gsem0, gsem1, ssem0, ssem1, lsem0, lsem1):
    c = lax.axis_index("c")
    s = lax.axis_index("s")
    wid = c * NS + s
    bounce = rows_v0

    def fill_zero(i, _):
      bounce[i] = jnp.zeros((F_OUT,), jnp.float32)
      return _
    lax.fori_loop(0, CH, fill_zero, None)

    pieces = [(z * CH, CH) for z in range(rpt // CH)]
    if rpt % CH:
      pieces.append((rpt - rpt % CH, rpt % CH))
    for off, ln in pieces:
      pltpu.sync_copy(bounce.at[pl.ds(0, ln)],
                      s_sh.at[pl.ds(s * rpt + off, ln)])
    plsc.subcore_barrier()

    def gath(pb, rows, sem, n=CRr):
      return [pltpu.async_copy(h2_hbm.at[pb.at[k, 0]],
                               rows.at[pl.ds(k * 128, 128)], sem)
              for k in range(n)]

    def scat(pb, rows, sem, n=CRr):
      for k in range(n):
        pltpu.async_copy(rows.at[pl.ds(k * 128, 128)],
                         s_sh.at[pb.at[k, 1]], sem, add=True)

    def drain(pb, rows, sem, n=CRr):
      for k in range(n):
        pltpu.make_async_copy(rows.at[pl.ds(k * 128, 128)],
                              s_sh.at[pb.at[k, 1]], sem).wait()

    def sup(q, _):
      g0 = (2 * q) * NW + wid
      g1 = (2 * q + 1) * NW + wid

      @pl.when(q > 0)
      def _():
        drain(pb0, rows_v0, ssem0)
      l0 = pltpu.async_copy(ei_hbm.at[pl.ds(g0 * CRr, CRr)], pb0, lsem0)

      @pl.when(q > 0)
      def _():
        drain(pb1, rows_v1, ssem1)
      l1 = pltpu.async_copy(ei_hbm.at[pl.ds(g1 * CRr, CRr)], pb1, lsem1)

      l0.wait()
      gd0 = gath(pb0, rows_v0, gsem0)
      l1.wait()
      gd1 = gath(pb1, rows_v1, gsem1)
      for d in gd0:
        d.wait()
      scat(pb0, rows_v0, ssem0)
      for d in gd1:
        d.wait()
      scat(pb1, rows_v1, ssem1)
      return _
    lax.fori_loop(0, nsup, sup, None)

    @pl.when(wid < extra)
    def _():
      drain(pb0, rows_v0, ssem0)
      pltpu.sync_copy(ei_hbm.at[pl.ds((base * NW + wid) * CRr, CRr)], pb0)
      for d in gath(pb0, rows_v0, gsem0):
        d.wait()
      scat(pb0, rows_v0, ssem0)

    if rem_rows:
      @pl.when(wid == NW - 1)
      def _():
        drain(pb1, rows_v1, ssem1)
        pltpu.sync_copy(ei_hbm.at[pl.ds(full_chunks * CRr, rem_rows)],
                        pb1.at[pl.ds(0, rem_rows)])
        for d in gath(pb1, rows_v1, gsem1, rem_rows):
          d.wait()
        scat(pb1, rows_v1, ssem1, rem_rows)

    drain(pb0, rows_v0, ssem0)
    if rem_rows:
      @pl.when(wid == NW - 1)
      def _():
        drain(pb1, rows_v1, ssem1, rem_rows)

      @pl.when(wid != NW - 1)
      def _():
        drain(pb1, rows_v1, ssem1)
    else:
      drain(pb1, rows_v1, ssem1)
    plsc.subcore_barrier()

    for off, ln in pieces:
      pltpu.sync_copy(s_sh.at[pl.ds(s * rpt + off, ln)],
                      bounce.at[pl.ds(0, ln)])

      @pl.when(c == 0)
      def _():
        pltpu.sync_copy(bounce.at[pl.ds(0, ln)],
                        out0.at[pl.ds(s * rpt + off, ln)])

      @pl.when(c == 1)
      def _():
        pltpu.sync_copy(bounce.at[pl.ds(0, ln)],
                        out1.at[pl.ds(s * rpt + off, ln)])

  return agg_k


def _mm_body(xt_ref, w_ref, h_ref):
  h_ref[...] = lax.dot_general(
      xt_ref[...], w_ref[...],
      dimension_numbers=(((0,), (0,)), ((), ())),
      preferred_element_type=jnp.float32)


def _scale_body(h_ref, d0_ref, d1_ref, h2_ref):
  dis = lax.rsqrt(d0_ref[...] + d1_ref[...] + 1.0)
  h2_ref[...] = h_ref[...] * dis[:, None]


def _fin_body(s0_ref, s1_ref, h2_ref, d0_ref, d1_ref, b_ref, o_ref):
  dis = lax.rsqrt(d0_ref[...] + d1_ref[...] + 1.0)
  t = s0_ref[...] + s1_ref[...] + h2_ref[...]
  o_ref[...] = jnp.maximum(dis[:, None] * t + b_ref[...], 0.0)


def kernel(X, edge_index, W1, b1):
  N, F_IN = X.shape
  F_OUT = W1.shape[1]
  E = edge_index.shape[1]

  NP = -(-N // 256) * 256
  R = E // 128
  ei3 = edge_index.reshape(2, R, 128).transpose(1, 0, 2)

  deg0, deg1 = _make_deg(R, NP)(ei3)

  BN = 2048
  nb = -(-N // BN)
  h = pl.pallas_call(
      _mm_body,
      grid=(nb,),
      in_specs=[
          pl.BlockSpec((F_IN, BN), lambda i: (0, i)),
          pl.BlockSpec((F_IN, F_OUT), lambda i: (0, 0)),
      ],
      out_specs=pl.BlockSpec((BN, F_OUT), lambda i: (i, 0)),
      out_shape=jax.ShapeDtypeStruct((N, F_OUT), jnp.float32),
  )(X.T, W1)

  BE = 8192
  ne = -(-NP // BE)
  h2 = pl.pallas_call(
      _scale_body,
      grid=(ne,),
      in_specs=[
          pl.BlockSpec((BE, F_OUT), lambda i: (i, 0)),
          pl.BlockSpec((BE,), lambda i: (i,)),
          pl.BlockSpec((BE,), lambda i: (i,)),
      ],
      out_specs=pl.BlockSpec((BE, F_OUT), lambda i: (i, 0)),
      out_shape=jax.ShapeDtypeStruct((NP, F_OUT), jnp.float32),
  )(h, deg0, deg1)

  s0, s1 = _make_agg(R, NP, F_OUT)(ei3, h2)

  out = pl.pallas_call(
      _fin_body,
      grid=(ne,),
      in_specs=[
          pl.BlockSpec((BE, F_OUT), lambda i: (i, 0)),
          pl.BlockSpec((BE, F_OUT), lambda i: (i, 0)),
          pl.BlockSpec((BE, F_OUT), lambda i: (i, 0)),
          pl.BlockSpec((BE,), lambda i: (i,)),
          pl.BlockSpec((BE,), lambda i: (i,)),
          pl.BlockSpec((1, F_OUT), lambda i: (0, 0)),
      ],
      out_specs=pl.BlockSpec((BE, F_OUT), lambda i: (i, 0)),
      out_shape=jax.ShapeDtypeStruct((N, F_OUT), jnp.float32),
  )(s0, s1, h2, deg0, deg1, b1[None, :])

  return out
